# baseline (speedup 1.0000x reference)
import functools

import jax
import jax.numpy as jnp
from jax import lax
from jax.experimental import pallas as pl
from jax.experimental.pallas import tpu as pltpu
from jax.experimental.pallas import tpu_sc as plsc

BATCH = 16384
EMBED = 32
NC, NS = 2, 16
NW = NC * NS
BPW = BATCH // NW
CHUNK = 128
NCH = BPW // CHUNK


def _gather_body(uidx_hbm, midx_hbm, ut_hbm, mt_hbm, ue_out, me_out,
                 uidx_v, midx_v, urows_v, mrows_v, sem_u, sem_m):
  wid = lax.axis_index("s") * NC + lax.axis_index("c")
  pltpu.sync_copy(uidx_hbm.at[wid], uidx_v)
  pltpu.sync_copy(midx_hbm.at[wid], midx_v)
  copies = []
  for j in range(NCH):
    dst = urows_v.at[pl.ds(j * CHUNK, CHUNK)]
    copies.append(pltpu.async_copy(ut_hbm.at[uidx_v.at[j]], dst, sem_u))
  for j in range(NCH):
    dst = mrows_v.at[pl.ds(j * CHUNK, CHUNK)]
    copies.append(pltpu.async_copy(mt_hbm.at[midx_v.at[j]], dst, sem_m))
  for c in copies:
    c.wait()
  base = wid * BPW
  pltpu.sync_copy(urows_v, ue_out.at[pl.ds(base, BPW)])
  pltpu.sync_copy(mrows_v, me_out.at[pl.ds(base, BPW)])


def _sc_gather(user, movie, user_table, movie_table):
  mesh = plsc.VectorSubcoreMesh(core_axis_name="c", subcore_axis_name="s")
  f = pl.kernel(
      _gather_body,
      out_type=[jax.ShapeDtypeStruct((BATCH, EMBED), jnp.float32),
                jax.ShapeDtypeStruct((BATCH, EMBED), jnp.float32)],
      mesh=mesh,
      scratch_types=[
          pltpu.VMEM((NCH, CHUNK), jnp.int32),
          pltpu.VMEM((NCH, CHUNK), jnp.int32),
          pltpu.VMEM((BPW, EMBED), jnp.float32),
          pltpu.VMEM((BPW, EMBED), jnp.float32),
          pltpu.SemaphoreType.DMA,
          pltpu.SemaphoreType.DMA,
      ],
      compiler_params=pltpu.CompilerParams(use_tc_tiling_on_sc=False),
  )
  u3 = user.astype(jnp.int32).reshape(NW, NCH, CHUNK)
  m3 = movie.astype(jnp.int32).reshape(NW, NCH, CHUNK)
  return f(u3, m3, user_table, movie_table)


TILE = 2048


def _mlp_body(ue, me, w0a, w0b, b0, w1, b1, w2, b2, w3, b3, out):
  x = jnp.maximum(ue[...] @ w0a[...] + me[...] @ w0b[...] + b0[...], 0.0)
  x = jnp.maximum(x @ w1[...] + b1[...], 0.0)
  x = jnp.maximum(x @ w2[...] + b2[...], 0.0)
  out[...] = x @ w3[...] + b3[...]


def _mlp(ue, me, W0, b0, W1, b1, W2, b2, W3, b3):
  full = lambda shape: pl.BlockSpec(shape, lambda i: (0, 0))
  return pl.pallas_call(
      _mlp_body,
      grid=(BATCH // TILE,),
      in_specs=[
          pl.BlockSpec((TILE, EMBED), lambda i: (i, 0)),
          pl.BlockSpec((TILE, EMBED), lambda i: (i, 0)),
          full((EMBED, 64)),
          full((EMBED, 64)),
          full((1, 64)),
          full((64, 32)),
          full((1, 32)),
          full((32, 16)),
          full((1, 16)),
          full((16, 1)),
          full((1, 1)),
      ],
      out_specs=pl.BlockSpec((TILE, 1), lambda i: (i, 0)),
      out_shape=jax.ShapeDtypeStruct((BATCH, 1), jnp.float32),
      compiler_params=pltpu.CompilerParams(
          dimension_semantics=("arbitrary",)),
  )(ue, me, W0[:EMBED], W0[EMBED:], b0.reshape(1, -1), W1,
    b1.reshape(1, -1), W2, b2.reshape(1, -1), W3, b3.reshape(1, -1))


def kernel(user, movie, user_table, movie_table, W0, b0, W1, b1, W2, b2, W3, b3):
  ue, me = _sc_gather(user, movie, user_table, movie_table)
  return _mlp(ue, me, W0, b0, W1, b1, W2, b2, W3, b3)
